# baseline (device time: 11213 ns/iter reference)
import jax
import jax.numpy as jnp
from jax import lax
from jax.experimental import pallas as pl
from jax.experimental.pallas import tpu as pltpu

N_DEV = 4
M = 256
H = M // 2
HH = H // 2
N_TOT = 1024
CHUNK = N_TOT // N_DEV


def kernel(x):
    x2 = x.reshape(M, N_TOT)

    def body(x_ref, out_ref, recv_a, recv_b, recv_a2, recv_b2, fwd_a, fwd_b,
             send_sems, recv_a_sems, recv_b_sems, recv_a2_sems, recv_b2_sems):
        my = lax.axis_index("i")
        p_y = my ^ 1
        p_x = my ^ 3
        diag = my ^ 2

        barrier_sem = pltpu.get_barrier_semaphore()
        for nbr in [p_y, p_x]:
            pl.semaphore_signal(
                barrier_sem, inc=1,
                device_id=(nbr,), device_id_type=pl.DeviceIdType.MESH,
            )
        pl.semaphore_wait(barrier_sem, 2)

        def rdma(src, dst, ssem, rsem, dev):
            return pltpu.make_async_remote_copy(
                src_ref=src, dst_ref=dst, send_sem=ssem, recv_sem=rsem,
                device_id=(dev,), device_id_type=pl.DeviceIdType.MESH,
            )

        def a_rows(r, n, c):
            return x_ref.at[pl.ds(r, n), pl.ds(c * CHUNK, CHUNK)]

        def b_rows(r, n, c):
            return x_ref.at[pl.ds(H + r, n), pl.ds(c * CHUNK, CHUNK)]

        a1_c0 = rdma(a_rows(0, HH, diag), recv_a.at[1, pl.ds(0, HH)],
                     send_sems.at[0], recv_a_sems.at[1], p_y)
        a1_c1 = rdma(a_rows(HH, HH, diag), recv_a.at[1, pl.ds(HH, HH)],
                     send_sems.at[1], recv_a_sems.at[2], p_y)
        a1_own = rdma(a_rows(0, H, p_y), recv_a.at[0],
                      send_sems.at[2], recv_a_sems.at[0], p_y)
        b1_c0 = rdma(b_rows(0, HH, diag), recv_b.at[1, pl.ds(0, HH)],
                     send_sems.at[3], recv_b_sems.at[1], p_x)
        b1_c1 = rdma(b_rows(HH, HH, diag), recv_b.at[1, pl.ds(HH, HH)],
                     send_sems.at[4], recv_b_sems.at[2], p_x)
        b1_own = rdma(b_rows(0, H, p_x), recv_b.at[0],
                      send_sems.at[5], recv_b_sems.at[0], p_x)

        a1_c0.start()
        b1_c0.start()
        a1_c1.start()
        b1_c1.start()
        a1_own.start()
        b1_own.start()

        a1_c0.wait_recv()
        fwd_a[pl.ds(0, HH), :] = (
            recv_a[1, pl.ds(0, HH), :]
            + x_ref[pl.ds(0, HH), pl.ds(p_x * CHUNK, CHUNK)]
        )
        a2_0 = rdma(fwd_a.at[pl.ds(0, HH)], recv_a2.at[pl.ds(0, HH)],
                    send_sems.at[6], recv_a2_sems.at[0], p_x)
        a2_0.start()
        b1_c0.wait_recv()
        fwd_b[pl.ds(0, HH), :] = (
            recv_b[1, pl.ds(0, HH), :]
            + x_ref[pl.ds(H, HH), pl.ds(p_y * CHUNK, CHUNK)]
        )
        b2_0 = rdma(fwd_b.at[pl.ds(0, HH)], recv_b2.at[pl.ds(0, HH)],
                    send_sems.at[7], recv_b2_sems.at[0], p_y)
        b2_0.start()

        a1_c1.wait_recv()
        fwd_a[pl.ds(HH, HH), :] = (
            recv_a[1, pl.ds(HH, HH), :]
            + x_ref[pl.ds(HH, HH), pl.ds(p_x * CHUNK, CHUNK)]
        )
        a2_1 = rdma(fwd_a.at[pl.ds(HH, HH)], recv_a2.at[pl.ds(HH, HH)],
                    send_sems.at[8], recv_a2_sems.at[1], p_x)
        a2_1.start()
        b1_c1.wait_recv()
        fwd_b[pl.ds(HH, HH), :] = (
            recv_b[1, pl.ds(HH, HH), :]
            + x_ref[pl.ds(H + HH, HH), pl.ds(p_y * CHUNK, CHUNK)]
        )
        b2_1 = rdma(fwd_b.at[pl.ds(HH, HH)], recv_b2.at[pl.ds(HH, HH)],
                    send_sems.at[9], recv_b2_sems.at[1], p_y)
        b2_1.start()

        a1_own.wait_recv()
        a2_0.wait_recv()
        a2_1.wait_recv()
        out_ref[pl.ds(0, H), :] = (
            x_ref[pl.ds(0, H), pl.ds(my * CHUNK, CHUNK)]
            + recv_a[0] + recv_a2[:, :]
        )
        b1_own.wait_recv()
        b2_0.wait_recv()
        b2_1.wait_recv()
        out_ref[pl.ds(H, H), :] = (
            x_ref[pl.ds(H, H), pl.ds(my * CHUNK, CHUNK)]
            + recv_b[0] + recv_b2[:, :]
        )

        for r in (a1_c0, a1_c1, a1_own, b1_c0, b1_c1, b1_own,
                  a2_0, a2_1, b2_0, b2_1):
            r.wait_send()

    return pl.pallas_call(
        body,
        out_shape=jax.ShapeDtypeStruct((M, CHUNK), jnp.float32),
        in_specs=[pl.BlockSpec(memory_space=pltpu.VMEM)],
        out_specs=pl.BlockSpec(memory_space=pltpu.VMEM),
        scratch_shapes=[
            pltpu.VMEM((2, H, CHUNK), jnp.float32),
            pltpu.VMEM((2, H, CHUNK), jnp.float32),
            pltpu.VMEM((H, CHUNK), jnp.float32),
            pltpu.VMEM((H, CHUNK), jnp.float32),
            pltpu.VMEM((H, CHUNK), jnp.float32),
            pltpu.VMEM((H, CHUNK), jnp.float32),
            pltpu.SemaphoreType.DMA((10,)),
            pltpu.SemaphoreType.DMA((3,)),
            pltpu.SemaphoreType.DMA((3,)),
            pltpu.SemaphoreType.DMA((2,)),
            pltpu.SemaphoreType.DMA((2,)),
        ],
        compiler_params=pltpu.CompilerParams(collective_id=0),
    )(x2)


# device time: 11115 ns/iter; 1.0088x vs baseline; 1.0088x over previous
import functools

import jax
import jax.numpy as jnp
from jax import lax
from jax.experimental import pallas as pl
from jax.experimental.pallas import tpu as pltpu

N_DEV = 4
M = 256
H = M // 2
N_TOT = 1024
CHUNK = N_TOT // N_DEV


def kernel(x):
    x2 = x.reshape(M, N_TOT)

    def body(x_ref, out_ref, recv_a, recv_b, recv_a2, recv_b2, fwd_a, fwd_b,
             send_sems, recv_a_sems, recv_b_sems, recv_a2_sem, recv_b2_sem):

        @functools.partial(pl.run_scoped, x_gate=pltpu.SemaphoreType.REGULAR)
        def _(x_gate):
            my = lax.axis_index("i")
            p_y = my ^ 1
            p_x = my ^ 3
            diag = my ^ 2

            barrier_sem = pltpu.get_barrier_semaphore()
            pl.semaphore_signal(
                barrier_sem, inc=1,
                device_id=(p_y,), device_id_type=pl.DeviceIdType.MESH,
            )
            pl.semaphore_signal(
                x_gate, inc=1,
                device_id=(p_x,), device_id_type=pl.DeviceIdType.MESH,
            )

            def rdma(src, dst, ssem, rsem, dev):
                return pltpu.make_async_remote_copy(
                    src_ref=src, dst_ref=dst, send_sem=ssem, recv_sem=rsem,
                    device_id=(dev,), device_id_type=pl.DeviceIdType.MESH,
                )

            def a_src(c):
                return x_ref.at[pl.ds(0, H), pl.ds(c * CHUNK, CHUNK)]

            def b_src(c):
                return x_ref.at[pl.ds(H, H), pl.ds(c * CHUNK, CHUNK)]

            a1_crit = rdma(a_src(diag), recv_a.at[1],
                           send_sems.at[0], recv_a_sems.at[1], p_y)
            a1_own = rdma(a_src(p_y), recv_a.at[0],
                          send_sems.at[2], recv_a_sems.at[0], p_y)
            b1_crit = rdma(b_src(diag), recv_b.at[1],
                           send_sems.at[1], recv_b_sems.at[1], p_x)
            b1_own = rdma(b_src(p_x), recv_b.at[0],
                          send_sems.at[3], recv_b_sems.at[0], p_x)

            pl.semaphore_wait(barrier_sem, 1)
            a1_crit.start()
            a1_own.start()
            pl.semaphore_wait(x_gate, 1)
            b1_crit.start()
            b1_own.start()

            a1_crit.wait_recv()
            fwd_a[:, :] = recv_a[1] + x_ref[pl.ds(0, H), pl.ds(p_x * CHUNK, CHUNK)]
            a2 = rdma(fwd_a, recv_a2, send_sems.at[4], recv_a2_sem, p_x)
            a2.start()

            b1_crit.wait_recv()
            fwd_b[:, :] = recv_b[1] + x_ref[pl.ds(H, H), pl.ds(p_y * CHUNK, CHUNK)]
            b2 = rdma(fwd_b, recv_b2, send_sems.at[5], recv_b2_sem, p_y)
            b2.start()

            a1_own.wait_recv()
            a2.wait_recv()
            out_ref[pl.ds(0, H), :] = (
                x_ref[pl.ds(0, H), pl.ds(my * CHUNK, CHUNK)]
                + recv_a[0] + recv_a2[:, :]
            )
            b1_own.wait_recv()
            b2.wait_recv()
            out_ref[pl.ds(H, H), :] = (
                x_ref[pl.ds(H, H), pl.ds(my * CHUNK, CHUNK)]
                + recv_b[0] + recv_b2[:, :]
            )

            for r in (a1_crit, b1_crit, a1_own, b1_own, a2, b2):
                r.wait_send()

    return pl.pallas_call(
        body,
        out_shape=jax.ShapeDtypeStruct((M, CHUNK), jnp.float32),
        in_specs=[pl.BlockSpec(memory_space=pltpu.VMEM)],
        out_specs=pl.BlockSpec(memory_space=pltpu.VMEM),
        scratch_shapes=[
            pltpu.VMEM((2, H, CHUNK), jnp.float32),
            pltpu.VMEM((2, H, CHUNK), jnp.float32),
            pltpu.VMEM((H, CHUNK), jnp.float32),
            pltpu.VMEM((H, CHUNK), jnp.float32),
            pltpu.VMEM((H, CHUNK), jnp.float32),
            pltpu.VMEM((H, CHUNK), jnp.float32),
            pltpu.SemaphoreType.DMA((6,)),
            pltpu.SemaphoreType.DMA((2,)),
            pltpu.SemaphoreType.DMA((2,)),
            pltpu.SemaphoreType.DMA,
            pltpu.SemaphoreType.DMA,
        ],
        compiler_params=pltpu.CompilerParams(collective_id=0),
    )(x2)
